# baseline (speedup 1.0000x reference)
import jax
import jax.numpy as jnp
from jax.experimental import pallas as pl
from jax.experimental.pallas import tpu as pltpu

_NUM_CLASSES = 80
_RADIUS = 2.5
_CAND_TOPK = 10
_IOU_W = 3.0
_CLS_W = 1.0
_SENT_SCALE = 2.0 ** 100
_SENT_INV = 2.0 ** -100


def _simota_kernel(scores_t_ref, pb_t_ref, ap_t_ref, gl_ref, gb_ref,
                   gtv_ref,
                   labels_out_ref, bboxes_t_out_ref, scores_t_out_ref,
                   cost_ref):
    C, A = scores_t_ref.shape
    G = gb_ref.shape[0]

    s = scores_t_ref[:, :]
    p = jax.nn.sigmoid(s)
    logp = jnp.maximum(jnp.log(p), -100.0)
    log1mp = jnp.maximum(jnp.log(1.0 - p), -100.0)
    sum_log1mp = jnp.sum(log1mp, axis=0, keepdims=True)
    diff = log1mp - logp
    labels = gl_ref[:, :]
    cls_iota = jax.lax.broadcasted_iota(jnp.int32, (G, C), 1)
    onehot = (cls_iota == labels).astype(jnp.float32)
    term = jax.lax.dot_general(
        onehot, diff, (((1,), (0,)), ((), ())),
        preferred_element_type=jnp.float32,
        precision=jax.lax.Precision.HIGHEST)
    cls_cost = term - sum_log1mp

    ax = ap_t_ref[0:1, :]
    ay = ap_t_ref[1:2, :]
    x1 = gb_ref[:, 0:1]
    y1 = gb_ref[:, 1:2]
    x2 = gb_ref[:, 2:3]
    y2 = gb_ref[:, 3:4]
    inside = (ax >= x1) & (ax <= x2) & (ay >= y1) & (ay <= y2)

    px1 = pb_t_ref[0:1, :]
    py1 = pb_t_ref[1:2, :]
    px2 = pb_t_ref[2:3, :]
    py2 = pb_t_ref[3:4, :]
    ltx = jnp.maximum(px1, x1)
    lty = jnp.maximum(py1, y1)
    rbx = jnp.minimum(px2, x2)
    rby = jnp.minimum(py2, y2)
    w = jnp.maximum(rbx - ltx, 0.0)
    h = jnp.maximum(rby - lty, 0.0)
    overlap = w * h
    area_p = (px2 - px1) * (py2 - py1)
    area_g = (x2 - x1) * (y2 - y1)
    union = area_p + area_g - overlap + 1e-6
    ious = overlap / union
    iou_cost = -jnp.log(ious)

    inside_f = inside.astype(jnp.float32)
    cost = (_CLS_W * cls_cost + _IOU_W * iou_cost
            + (1.0 - inside_f) * 1e10)

    nc = jnp.sum(((ious * inside_f) > 0).astype(jnp.int32),
                 axis=1, keepdims=True)
    ks = jnp.clip(nc, 1, _CAND_TOPK)

    a_iota = jax.lax.broadcasted_iota(jnp.int32, (G, A), 1)

    cost_ref[:, :] = cost

    def body(t, _):
        cost_c = cost_ref[:, :]
        m = jnp.min(cost_c, axis=1, keepdims=True)
        idx = jnp.min(jnp.where(cost_c == m, a_iota, jnp.int32(A)),
                      axis=1, keepdims=True)
        pick = a_iota == idx
        snt = (t + 2).astype(jnp.float32) * _SENT_SCALE
        cost_ref[:, :] = jnp.where(
            pick,
            jnp.where(cost_c >= _SENT_SCALE, cost_c, snt),
            cost_c)
        return 0

    jax.lax.fori_loop(0, _CAND_TOPK, body, 0)

    tdec = cost_ref[:, :] * _SENT_INV
    ksf = ks.astype(jnp.float32) + 2.0
    sel = (tdec >= 2.0) & (tdec < ksf)

    g_iota = jax.lax.broadcasted_iota(jnp.int32, (G, A), 0)
    assigned = jnp.max(jnp.where(sel, g_iota, -1),
                       axis=0, keepdims=True)
    pos = assigned >= 0
    oh = g_iota == assigned
    ohf = oh.astype(jnp.float32)

    iou_val = jnp.sum(ious * ohf, axis=0, keepdims=True)
    rows5 = jax.lax.dot_general(
        gtv_ref[:, :], ohf, (((1,), (0,)), ((), ())),
        preferred_element_type=jnp.float32,
        precision=jax.lax.Precision.HIGHEST)
    lbl = rows5[0:1, :].astype(jnp.int32)
    labels_out_ref[:, :] = jnp.where(pos, lbl, _NUM_CLASSES)
    bboxes_t_out_ref[:, :] = jnp.where(pos, rows5[1:5, :], 0.0)

    colid = jnp.where(pos, lbl, _NUM_CLASSES)
    val = jnp.where(pos, iou_val, 0.0)
    c_iota = jax.lax.broadcasted_iota(
        jnp.int32, (_NUM_CLASSES + 1, A), 0)
    scores_t_out_ref[:, :] = jnp.where(c_iota == colid, val, 0.0)


def kernel(pred_scores, pred_bboxes, anchor_points, gt_labels, gt_bboxes):
    A, C = pred_scores.shape
    G = gt_bboxes.shape[0]
    scores_t = pred_scores.T
    pb_t = pred_bboxes.T
    ap_t = anchor_points.T
    gl = gt_labels.reshape(G, 1).astype(jnp.int32)
    gb = gt_bboxes.astype(jnp.float32)
    gtv = jnp.concatenate(
        [gl.astype(jnp.float32).T, gb.T], axis=0)

    labels_t, bboxes_t, scores_out_t = pl.pallas_call(
        _simota_kernel,
        out_shape=(
            jax.ShapeDtypeStruct((1, A), jnp.int32),
            jax.ShapeDtypeStruct((4, A), jnp.float32),
            jax.ShapeDtypeStruct((_NUM_CLASSES + 1, A), jnp.float32),
        ),
        scratch_shapes=[
            pltpu.VMEM((G, A), jnp.float32),
        ],
    )(scores_t, pb_t, ap_t, gl, gb, gtv)

    return labels_t.reshape(A), bboxes_t.T, scores_out_t.T
